# baseline (speedup 1.0000x reference)
import functools

import jax
import jax.numpy as jnp
from jax import lax
from jax.experimental import pallas as pl
from jax.experimental.pallas import tpu as pltpu
from jax.experimental.pallas import tpu_sc as plsc

D = 768
FF = 3072
E = 8
K = 2
T = 2048

TILE_M = 512
TSH = 9
P = T * K + E * TILE_M
NT = P // TILE_M
TF = 1536
NF = FF // TF

_BT1 = 256
_NB1 = T // _BT1
_BT4 = 256

NW = 32
TPW = T // NW


def _k1_body(x_ref, wg_ref, g_ref, b_ref, h_ref, ti_ref, tw_ref,
             pos_ref, meta_ref, ef_s):
    i = pl.program_id(0)
    n = T * K

    @pl.when(i < _NB1)
    def _ln_router():
        x = x_ref[...]
        mu = jnp.mean(x, axis=-1, keepdims=True)
        var = jnp.mean((x - mu) ** 2, axis=-1, keepdims=True)
        h = (x - mu) / jnp.sqrt(var + 1e-5) * g_ref[...] + b_ref[...]
        h_ref[...] = h
        logits = jnp.dot(h, wg_ref[...], preferred_element_type=jnp.float32)
        m = jnp.max(logits, axis=-1, keepdims=True)
        ex = jnp.exp(logits - m)
        p = ex / jnp.sum(ex, axis=-1, keepdims=True)
        ei = jax.lax.broadcasted_iota(jnp.int32, (_BT1, E), 1)
        v1 = jnp.max(p, axis=-1, keepdims=True)
        i1 = jnp.min(jnp.where(p == v1, ei, E), axis=-1, keepdims=True)
        p2 = jnp.where(ei == i1, -1.0, p)
        v2 = jnp.max(p2, axis=-1, keepdims=True)
        i2 = jnp.min(jnp.where(p2 == v2, ei, E), axis=-1, keepdims=True)
        den = v1 + v2 + 1e-9
        ti_ref[...] = jnp.concatenate([i1, i2], axis=1)
        tw_ref[...] = jnp.concatenate([v1 / den, v2 / den], axis=1)
        ef_s[pl.ds(i * _BT1, _BT1), :] = i1
        ef_s[pl.ds(T + i * _BT1, _BT1), :] = i2

    @pl.when(i == _NB1)
    def _route():
        e = ef_s[...]
        se = jax.lax.broadcasted_iota(jnp.int32, (n, E), 1)
        oh = (e == se).astype(jnp.int32)
        c = oh
        s = 1
        while s < n:
            z = jnp.zeros((s, E), jnp.int32)
            c = c + jnp.concatenate([z, c[: n - s, :]], axis=0)
            s *= 2
        rank = jnp.sum(oh * c, axis=1, keepdims=True) - 1
        counts = c[n - 1 : n, :]
        psz = ((counts + (TILE_M - 1)) >> TSH) << TSH
        q = psz
        s = 1
        while s < E:
            z = jnp.zeros((1, s), jnp.int32)
            q = q + jnp.concatenate([z, q[:, : E - s]], axis=1)
            s *= 2
        off = q - psz
        pos_ref[...] = jnp.sum(oh * off, axis=1, keepdims=True) + rank
        p_used = q[:, E - 1 : E]
        ti = jax.lax.broadcasted_iota(jnp.int32, (1, 64), 1) * TILE_M
        texp = jnp.zeros((1, 64), jnp.int32)
        for ee in range(1, E):
            texp = texp + (ti >= off[:, ee : ee + 1]).astype(jnp.int32)
        nv = p_used >> TSH
        li = jax.lax.broadcasted_iota(jnp.int32, (1, 64), 1)
        meta_ref[...] = jnp.where(li == 63, nv, texp)


def _sca_body(pos_hbm, h_hbm, hs_hbm, p0_v, p1_v, hv, sem):
    wid = lax.axis_index("s") * 2 + lax.axis_index("c")
    t0 = wid * TPW
    pltpu.sync_copy(pos_hbm.at[pl.ds(t0, TPW)], p0_v)
    pltpu.sync_copy(pos_hbm.at[pl.ds(T + t0, TPW)], p1_v)
    pltpu.sync_copy(h_hbm.at[pl.ds(t0, TPW)], hv)
    pltpu.async_copy(hv, hs_hbm.at[p0_v], sem).wait()
    pltpu.async_copy(hv, hs_hbm.at[p1_v], sem).wait()


def _scb_body(pos_hbm, ys_hbm, ysg_hbm, p0_v, p1_v, rows_v, sem):
    wid = lax.axis_index("s") * 2 + lax.axis_index("c")
    t0 = wid * TPW
    pltpu.sync_copy(pos_hbm.at[pl.ds(t0, TPW)], p0_v)
    pltpu.sync_copy(pos_hbm.at[pl.ds(T + t0, TPW)], p1_v)
    pltpu.async_copy(ys_hbm.at[p0_v], rows_v, sem).wait()
    pltpu.sync_copy(rows_v, ysg_hbm.at[pl.ds(t0, TPW)])
    pltpu.async_copy(ys_hbm.at[p1_v], rows_v, sem).wait()
    pltpu.sync_copy(rows_v, ysg_hbm.at[pl.ds(T + t0, TPW)])


def _k3_body(meta_ref, hs_ref, w1_ref, b1_ref, w2_ref, b2_ref, ys_ref):
    i = pl.program_id(0)
    f = pl.program_id(1)
    nv = meta_ref[63]

    @pl.when(i < nv)
    def _ffn():
        hs_bf = hs_ref[...].astype(jnp.bfloat16)
        a = jnp.maximum(
            jnp.dot(hs_bf, w1_ref[0].astype(jnp.bfloat16),
                    preferred_element_type=jnp.float32) + b1_ref[0], 0.0)
        contrib = jnp.dot(a.astype(jnp.bfloat16),
                          w2_ref[0].astype(jnp.bfloat16),
                          preferred_element_type=jnp.float32)

        @pl.when(f == 0)
        def _init():
            ys_ref[...] = b2_ref[0] + contrib

        @pl.when(f > 0)
        def _acc():
            ys_ref[...] = ys_ref[...] + contrib


def _k4_body(x_ref, w_ref, y0_ref, y1_ref, o_ref):
    o_ref[...] = (x_ref[...] + w_ref[:, 0:1] * y0_ref[0]
                  + w_ref[:, 1:2] * y1_ref[0])


def kernel(x, Wg, W1, b1, W2, b2, gamma, beta):
    g2 = gamma.reshape(1, D)
    bt2 = beta.reshape(1, D)

    h, tidx, tw, pos1, meta = pl.pallas_call(
        _k1_body,
        grid=(_NB1 + 1,),
        in_specs=[
            pl.BlockSpec((_BT1, D), lambda i: (jnp.minimum(i, _NB1 - 1), 0)),
            pl.BlockSpec((D, E), lambda i: (0, 0)),
            pl.BlockSpec((1, D), lambda i: (0, 0)),
            pl.BlockSpec((1, D), lambda i: (0, 0)),
        ],
        out_specs=[
            pl.BlockSpec((_BT1, D), lambda i: (jnp.minimum(i, _NB1 - 1), 0)),
            pl.BlockSpec((_BT1, K), lambda i: (jnp.minimum(i, _NB1 - 1), 0)),
            pl.BlockSpec((_BT1, K), lambda i: (jnp.minimum(i, _NB1 - 1), 0)),
            pl.BlockSpec((T * K, 1), lambda i: (0, 0)),
            pl.BlockSpec((1, 64), lambda i: (0, 0)),
        ],
        out_shape=[
            jax.ShapeDtypeStruct((T, D), jnp.float32),
            jax.ShapeDtypeStruct((T, K), jnp.int32),
            jax.ShapeDtypeStruct((T, K), jnp.float32),
            jax.ShapeDtypeStruct((T * K, 1), jnp.int32),
            jax.ShapeDtypeStruct((1, 64), jnp.int32),
        ],
        scratch_shapes=[pltpu.VMEM((T * K, 1), jnp.int32)],
        compiler_params=pltpu.CompilerParams(
            dimension_semantics=("arbitrary",)),
    )(x, Wg, g2, bt2)

    posf = pos1.reshape(T * K)
    meta1 = meta.reshape(64)

    sca = pl.kernel(
        _sca_body,
        out_type=jax.ShapeDtypeStruct((P, D), jnp.float32),
        mesh=plsc.VectorSubcoreMesh(core_axis_name="c", subcore_axis_name="s"),
        compiler_params=pltpu.CompilerParams(needs_layout_passes=False),
        scratch_types=[
            pltpu.VMEM((TPW,), jnp.int32),
            pltpu.VMEM((TPW,), jnp.int32),
            pltpu.VMEM((TPW, D), jnp.float32),
            pltpu.SemaphoreType.DMA,
        ],
    )
    hs = sca(posf, h)

    ys = pl.pallas_call(
        _k3_body,
        grid_spec=pltpu.PrefetchScalarGridSpec(
            num_scalar_prefetch=1,
            grid=(NT, NF),
            in_specs=[
                pl.BlockSpec((TILE_M, D), lambda i, f, m: (i, 0)),
                pl.BlockSpec((1, D, TF), lambda i, f, m: (m[i], 0, f)),
                pl.BlockSpec((1, 1, TF), lambda i, f, m: (m[i], 0, f)),
                pl.BlockSpec((1, TF, D), lambda i, f, m: (m[i], f, 0)),
                pl.BlockSpec((1, 1, D), lambda i, f, m: (m[i], 0, 0)),
            ],
            out_specs=pl.BlockSpec((TILE_M, D), lambda i, f, m: (i, 0)),
        ),
        out_shape=jax.ShapeDtypeStruct((P, D), jnp.float32),
        compiler_params=pltpu.CompilerParams(
            dimension_semantics=("arbitrary", "arbitrary")),
    )(meta1, hs, W1, b1.reshape(E, 1, FF), W2, b2.reshape(E, 1, D))

    scb = pl.kernel(
        _scb_body,
        out_type=jax.ShapeDtypeStruct((T * K, D), jnp.float32),
        mesh=plsc.VectorSubcoreMesh(core_axis_name="c", subcore_axis_name="s"),
        compiler_params=pltpu.CompilerParams(needs_layout_passes=False),
        scratch_types=[
            pltpu.VMEM((TPW,), jnp.int32),
            pltpu.VMEM((TPW,), jnp.int32),
            pltpu.VMEM((TPW, D), jnp.float32),
            pltpu.SemaphoreType.DMA,
        ],
    )
    ysg = scb(posf, ys)
    ysg3 = ysg.reshape(K, T, D)

    out = pl.pallas_call(
        _k4_body,
        grid=(T // _BT4,),
        in_specs=[
            pl.BlockSpec((_BT4, D), lambda i: (i, 0)),
            pl.BlockSpec((_BT4, K), lambda i: (i, 0)),
            pl.BlockSpec((1, _BT4, D), lambda i: (0, i, 0)),
            pl.BlockSpec((1, _BT4, D), lambda i: (1, i, 0)),
        ],
        out_specs=pl.BlockSpec((_BT4, D), lambda i: (i, 0)),
        out_shape=jax.ShapeDtypeStruct((T, D), jnp.float32),
    )(x, tw, ysg3, ysg3)

    return out
